# baseline (speedup 1.0000x reference)
import functools

import jax
import jax.numpy as jnp
from jax import lax
from jax.experimental import pallas as pl
from jax.experimental.pallas import tpu as pltpu
from jax.experimental.pallas import tpu_sc as plsc

NN = 100000
NE = 1600000
HD = 32
KK = 30
NB = 512
EPSV = 1e-5

BR = 4096
GRID = 25
NR = GRID * BR
EP = 1601536
CH = 128
NHALF = NN // 2
TRASH = NHALF
SENT = NN
NC, NS = 2, 16
NVBUF = 100864
NH2 = 50176

_MESH = plsc.VectorSubcoreMesh(core_axis_name="c", subcore_axis_name="s")


@functools.partial(
    pl.kernel,
    out_type=(
        jax.ShapeDtypeStruct((NR,), jnp.float32),
        jax.ShapeDtypeStruct((2, NB), jnp.float32),
        jax.ShapeDtypeStruct((NR, HD), jnp.float32),
    ),
    mesh=_MESH,
    compiler_params=pltpu.CompilerParams(use_tc_tiling_on_sc=False, needs_layout_passes=False),
    scratch_types=[
        pltpu.VMEM((CH,), jnp.int32),
        pltpu.VMEM((CH,), jnp.int32),
        pltpu.VMEM((CH,), jnp.float32),
        pltpu.VMEM((CH, HD), jnp.float32),
        pltpu.VMEM_SHARED((51200,), jnp.float32),
        pltpu.VMEM_SHARED((NB,), jnp.float32),
        pltpu.SemaphoreType.DMA,
    ],
)
def _sc_prep(dst_hbm, batch_hbm, x_hbm, zemb_hbm, zeros_hbm,
             deg_hbm, counts_hbm, h0_hbm,
             idxb, lidxb, valb, rowsb, deg_sh, cnt_sh, sem):
    cid = lax.axis_index("c")
    sid = lax.axis_index("s")
    base = cid * 51200

    pltpu.sync_copy(zeros_hbm.at[pl.ds(0, 3200)],
                    deg_sh.at[pl.ds(sid * 3200, 3200)])

    @pl.when(sid == 0)
    def _():
        pltpu.sync_copy(zeros_hbm.at[pl.ds(0, NB)], cnt_sh)

    plsc.subcore_barrier()

    ept = EP // NS
    @pl.loop(0, ept // CH)
    def _(k):
        off = sid * ept + k * CH
        pltpu.sync_copy(dst_hbm.at[pl.ds(off, CH)], idxb)
        for j in range(CH // 16):
            dd = idxb[pl.ds(j * 16, 16)]
            m = (dd >= base) & (dd < base + 51200)
            lidxb[pl.ds(j * 16, 16)] = jnp.where(m, dd - base, 0)
            valb[pl.ds(j * 16, 16)] = jnp.where(m, 1.0, 0.0)
        pltpu.sync_copy(valb, deg_sh.at[lidxb], add=True)

    @pl.loop(0, 25)
    def _(k):
        off = cid * 51200 + sid * 3200 + k * CH
        pltpu.sync_copy(batch_hbm.at[pl.ds(off, CH)], idxb)
        for j in range(CH // 16):
            bb = idxb[pl.ds(j * 16, 16)]
            m = bb >= 0
            lidxb[pl.ds(j * 16, 16)] = jnp.where(m, bb, 0)
            valb[pl.ds(j * 16, 16)] = jnp.where(m, 1.0, 0.0)
        pltpu.sync_copy(valb, cnt_sh.at[lidxb], add=True)

    wid = sid * NC + cid
    @pl.loop(0, 25)
    def _(k):
        off = wid * 3200 + k * CH
        pltpu.sync_copy(x_hbm.at[pl.ds(off, CH)], idxb)
        pltpu.async_copy(zemb_hbm.at[idxb], rowsb, sem).wait()
        pltpu.sync_copy(rowsb, h0_hbm.at[pl.ds(off, CH)])

    plsc.subcore_barrier()
    pltpu.sync_copy(deg_sh.at[pl.ds(sid * 3200, 3200)],
                    deg_hbm.at[pl.ds(base + sid * 3200, 3200)])

    @pl.when(sid == 0)
    def _():
        pltpu.sync_copy(cnt_sh, counts_hbm.at[cid])


@functools.partial(
    pl.kernel,
    out_type=jax.ShapeDtypeStruct((EP,), jnp.float32),
    mesh=_MESH,
    compiler_params=pltpu.CompilerParams(use_tc_tiling_on_sc=False, needs_layout_passes=False),
    scratch_types=[
        pltpu.VMEM((CH,), jnp.int32),
        pltpu.VMEM((CH,), jnp.int32),
        pltpu.VMEM((CH,), jnp.float32),
        pltpu.VMEM((NR,), jnp.float32),
        pltpu.SemaphoreType.DMA,
    ],
)
def _sc_w(src_hbm, dst_hbm, dinv_hbm, w_hbm, srcb, dstb, wb, dinvb, sem):
    cid = lax.axis_index("c")
    sid = lax.axis_index("s")
    wid = sid * NC + cid

    pltpu.sync_copy(dinv_hbm, dinvb)
    ept = EP // (NC * NS)
    @pl.loop(0, ept // CH)
    def _(k):
        off = wid * ept + k * CH
        cs = pltpu.async_copy(src_hbm.at[pl.ds(off, CH)], srcb, sem)
        cd = pltpu.async_copy(dst_hbm.at[pl.ds(off, CH)], dstb, sem)
        cs.wait()
        cd.wait()
        for j in range(CH // 16):
            dd = jnp.maximum(dstb[pl.ds(j * 16, 16)], 0)
            ss = srcb[pl.ds(j * 16, 16)]
            wb[pl.ds(j * 16, 16)] = (plsc.load_gather(dinvb, [ss])
                                     * plsc.load_gather(dinvb, [dd]))
        pltpu.sync_copy(wb, w_hbm.at[pl.ds(off, CH)])


NOWN = NR // 32
PCAP = 272


@functools.partial(
    pl.kernel,
    out_type=jax.ShapeDtypeStruct((NR, HD), jnp.float32),
    mesh=_MESH,
    compiler_params=pltpu.CompilerParams(use_tc_tiling_on_sc=False, needs_layout_passes=False),
    scratch_types=[
        pltpu.VMEM((CH,), jnp.int32),
        pltpu.VMEM((CH,), jnp.int32),
        pltpu.VMEM((CH,), jnp.float32),
        pltpu.VMEM((PCAP,), jnp.int32),
        pltpu.VMEM((PCAP,), jnp.int32),
        pltpu.VMEM((PCAP,), jnp.float32),
        pltpu.VMEM((PCAP, HD), jnp.float32),
        pltpu.VMEM((NOWN, HD), jnp.float32),
        pltpu.SemaphoreType.DMA,
    ],
)
def _sc_edges(src_hbm, dst_hbm, hs_hbm, w_hbm, zeros2_hbm, acc_hbm,
              srcb, dstb, wb, psrc, plidx, pw, rowsb, accb, sem):
    cid = lax.axis_index("c")
    sid = lax.axis_index("s")
    wid = sid * NC + cid
    base = wid * NOWN

    pltpu.sync_copy(zeros2_hbm.at[pl.ds(0, NOWN)], accb)

    @pl.loop(0, PCAP // 16)
    def _(i):
        psrc[pl.ds(i * 16, 16)] = jnp.zeros((16,), jnp.int32)

    def flush(pcount):
        pltpu.async_copy(hs_hbm.at[psrc], rowsb, sem).wait()

        @pl.loop(0, pcount)
        def _(e):
            li = plidx[pl.ds(e, 16)][0]
            wf = jnp.full((16,), pw[pl.ds(e, 16)][0])
            accb[li, pl.ds(0, 16)] = (accb[li, pl.ds(0, 16)]
                                      + rowsb[e, pl.ds(0, 16)] * wf)
            accb[li, pl.ds(16, 16)] = (accb[li, pl.ds(16, 16)]
                                       + rowsb[e, pl.ds(16, 16)] * wf)

    @pl.loop(0, EP // CH, init_carry=jnp.int32(0))
    def _(k, pcount):
        off = k * CH
        cs = pltpu.async_copy(src_hbm.at[pl.ds(off, CH)], srcb, sem)
        cd = pltpu.async_copy(dst_hbm.at[pl.ds(off, CH)], dstb, sem)
        cw = pltpu.async_copy(w_hbm.at[pl.ds(off, CH)], wb, sem)
        cs.wait()
        cd.wait()
        cw.wait()
        for j in range(CH // 16):
            dd = dstb[pl.ds(j * 16, 16)]
            m = (dd >= base) & (dd < base + NOWN)
            plsc.store_compressed(psrc.at[pl.ds(pcount, 16)],
                                  srcb[pl.ds(j * 16, 16)], mask=m)
            plsc.store_compressed(plidx.at[pl.ds(pcount, 16)], dd - base,
                                  mask=m)
            plsc.store_compressed(pw.at[pl.ds(pcount, 16)],
                                  wb[pl.ds(j * 16, 16)], mask=m)
            pcount = pcount + plsc.all_reduce_population_count(m)[0]

        @pl.when(pcount >= CH)
        def _():
            flush(pcount)

        return jnp.where(pcount >= CH, 0, pcount)

    pcount_last = _
    flush(pcount_last)

    pltpu.sync_copy(accb, acc_hbm.at[pl.ds(base, NOWN)])


@functools.partial(
    pl.kernel,
    out_type=jax.ShapeDtypeStruct((32, 512), jnp.int32),
    mesh=_MESH,
    compiler_params=pltpu.CompilerParams(use_tc_tiling_on_sc=False, needs_layout_passes=False),
    scratch_types=[
        pltpu.VMEM((NVBUF,), jnp.float32),
        pltpu.VMEM((NB,), jnp.float32),
        pltpu.VMEM((NB,), jnp.float32),
        pltpu.VMEM((NB + 32,), jnp.int32),
        pltpu.VMEM((NB + 32,), jnp.int32),
        pltpu.VMEM((NB,), jnp.int32),
        pltpu.SemaphoreType.DMA,
    ],
)
def _sc_rank(vals_hbm, cs_hbm, nodeof_hbm,
             vbuf, cp0, cp1, cntb, startsb, slotb, sem):
    cid = lax.axis_index("c")
    sid = lax.axis_index("s")
    wid = sid * NC + cid

    pltpu.sync_copy(cs_hbm.at[0], cp0)
    pltpu.sync_copy(cs_hbm.at[1], cp1)

    @pl.loop(0, NB // 16)
    def _(i):
        cntb[pl.ds(i * 16, 16)] = cp0[pl.ds(i * 16, 16)].astype(jnp.int32)
        startsb[pl.ds(i * 16, 16)] = cp1[pl.ds(i * 16, 16)].astype(jnp.int32)

    lanes = lax.iota(jnp.int32, 16)

    @pl.loop(0, NB // 32)
    def _(t):
        g = t * 32 + wid
        st = startsb[pl.ds(g, 16)][0]
        c = cntb[pl.ds(g, 16)][0]
        s8 = (st // 8) * 8
        off = st - s8
        nch = (off + c + 511) // 512

        @pl.loop(0, nch)
        def _(k):
            pltpu.sync_copy(vals_hbm.at[pl.ds(s8 + k * 512, 512)],
                            vbuf.at[pl.ds(k * 512, 512)])

        slotb[pl.ds(t * 32, 16)] = jnp.full((16,), SENT, jnp.int32)
        slotb[pl.ds(t * 32 + 16, 16)] = jnp.full((16,), SENT, jnp.int32)

        @pl.loop(0, (c + 15) // 16)
        def _(ic):
            ivec = lanes + ic * 16
            vi = vbuf[pl.ds(off + ic * 16, 16)]

            def body_jc(jc, rank):
                vj16 = vbuf[pl.ds(off + jc * 16, 16)]
                for jj in range(16):
                    vjs = jnp.full((16,), vj16[jj])
                    jidx = jc * 16 + jj
                    valid = jidx < c
                    gt = (vjs > vi) & valid
                    eq = (vjs == vi) & (jidx < ivec) & valid
                    rank = rank + jnp.where(gt, 1, 0) + jnp.where(eq, 1, 0)
                return rank

            rank = lax.fori_loop(0, (c + 15) // 16, body_jc,
                                 jnp.zeros((16,), jnp.int32))
            m = (ivec < c) & (rank < KK)
            plsc.store_scatter(slotb, [t * 32 + jnp.minimum(rank, 31)],
                               st + ivec, mask=m)

    pltpu.sync_copy(slotb, nodeof_hbm.at[wid])


@functools.partial(
    pl.kernel,
    out_type=jax.ShapeDtypeStruct((NB * 32, HD), jnp.float32),
    mesh=_MESH,
    compiler_params=pltpu.CompilerParams(use_tc_tiling_on_sc=False, needs_layout_passes=False),
    scratch_types=[
        pltpu.VMEM((CH,), jnp.int32),
        pltpu.VMEM((CH, HD), jnp.float32),
        pltpu.SemaphoreType.DMA,
    ],
)
def _sc_pool(hext_hbm, nodeof_hbm, pooled_hbm, idxb, rowsb, sem):
    cid = lax.axis_index("c")
    sid = lax.axis_index("s")
    wid = sid * NC + cid

    @pl.loop(0, (NB * 32) // (32 * CH))
    def _(k):
        off = wid * 512 + k * CH
        pltpu.sync_copy(nodeof_hbm.at[pl.ds(off, CH)], idxb)
        pltpu.async_copy(hext_hbm.at[idxb], rowsb, sem).wait()
        pltpu.sync_copy(rowsb, pooled_hbm.at[pl.ds(off, CH)])


def _row_spec():
    return pl.BlockSpec((BR, HD), lambda i: (i, 0))


def _col_spec():
    return pl.BlockSpec((BR, 1), lambda i: (i, 0))


def _full_spec(shape):
    return pl.BlockSpec(shape, lambda i: tuple(0 for _ in shape))


def _tc_dinv_body(deg_ref, dinv_ref):
    dinv_ref[...] = lax.rsqrt(deg_ref[...] + 1.0)


def _tc_m0_body(h0_ref, w_ref, hs_ref):
    hs_ref[...] = jnp.dot(h0_ref[...], w_ref[...],
                          preferred_element_type=jnp.float32)


def _tc_t_body(deg_ref, acc_ref, hp_ref, b_ref, t_ref, s1_ref):
    i = pl.program_id(0)
    dinv = lax.rsqrt(deg_ref[...] + 1.0)
    t = acc_ref[...] + hp_ref[...] * (dinv * dinv) + b_ref[...]
    t_ref[...] = t
    rows = lax.broadcasted_iota(jnp.int32, (BR, 1), 0) + i * BR
    tm = jnp.where(rows < NN, t, 0.0)

    @pl.when(i == 0)
    def _():
        s1_ref[...] = jnp.zeros((8, HD), jnp.float32)

    s1_ref[0:1, :] = s1_ref[0:1, :] + jnp.sum(tm, axis=0, keepdims=True)


def _tc_v_body(t_ref, s1_ref, s2_ref):
    i = pl.program_id(0)
    mean = s1_ref[0:1, :] / NN
    dev = t_ref[...] - mean
    rows = lax.broadcasted_iota(jnp.int32, (BR, 1), 0) + i * BR
    dev = jnp.where(rows < NN, dev, 0.0)

    @pl.when(i == 0)
    def _():
        s2_ref[...] = jnp.zeros((8, HD), jnp.float32)

    s2_ref[0:1, :] = s2_ref[0:1, :] + jnp.sum(dev * dev, axis=0, keepdims=True)


def _tc_m_body(t_ref, s1_ref, s2_ref, g_ref, be_ref, w_ref, hs_ref):
    mean = s1_ref[0:1, :] / NN
    var = s2_ref[0:1, :] / NN
    y = (t_ref[...] - mean) * lax.rsqrt(var + EPSV) * g_ref[...] + be_ref[...]
    y = jnp.maximum(y, 0.0)
    hs_ref[...] = jnp.dot(y, w_ref[...], preferred_element_type=jnp.float32)


def _tc_y_body(t_ref, s1_ref, s2_ref, g_ref, be_ref, hext_ref, vals_ref):
    i = pl.program_id(0)
    mean = s1_ref[0:1, :] / NN
    var = s2_ref[0:1, :] / NN
    y = (t_ref[...] - mean) * lax.rsqrt(var + EPSV) * g_ref[...] + be_ref[...]
    y = jnp.maximum(y, 0.0)
    rows = lax.broadcasted_iota(jnp.int32, (BR, 1), 0) + i * BR
    y = jnp.where(rows < NN, y, 0.0)
    hext_ref[...] = y
    vals_ref[...] = y[:, HD - 1:HD]


def _tc_starts_body(c_ref, cs_ref):
    cnt = c_ref[0:1, :] + c_ref[1:2, :]
    ii = lax.broadcasted_iota(jnp.int32, (NB, NB), 0)
    jj = lax.broadcasted_iota(jnp.int32, (NB, NB), 1)
    lt = jnp.where(ii < jj, 1.0, 0.0)
    starts = jnp.dot(cnt, lt, preferred_element_type=jnp.float32)
    cs_ref[0:1, :] = cnt
    cs_ref[1:2, :] = starts


def _tc_starts(counts):
    return pl.pallas_call(
        _tc_starts_body,
        out_shape=jax.ShapeDtypeStruct((2, NB), jnp.float32),
    )(counts)


def _tc_mlp_body(p_ref, w1_ref, b1_ref, w2_ref, b2_ref, w3t_ref, b3_ref, o_ref):
    h2 = jnp.dot(p_ref[...], w1_ref[...], preferred_element_type=jnp.float32)
    h2 = jnp.maximum(h2 + b1_ref[...], 0.0)
    h3 = jnp.dot(h2, w2_ref[...], preferred_element_type=jnp.float32)
    h3 = jnp.maximum(h3 + b2_ref[...], 0.0)
    o = jnp.sum(h3 * w3t_ref[...], axis=1, keepdims=True) + b3_ref[...]
    o_ref[...] = o


def _tc_dinv(deg25):
    return pl.pallas_call(
        _tc_dinv_body,
        out_shape=jax.ShapeDtypeStruct((GRID, BR), jnp.float32),
    )(deg25)


def _tc_m0(h0, w):
    return pl.pallas_call(
        _tc_m0_body,
        grid=(GRID,),
        in_specs=[_row_spec(), _full_spec((HD, HD))],
        out_specs=_row_spec(),
        out_shape=jax.ShapeDtypeStruct((NR, HD), jnp.float32),
    )(h0, w)


def _tc_t(deg2d, acc, hp, b):
    return pl.pallas_call(
        _tc_t_body,
        grid=(GRID,),
        in_specs=[_col_spec(), _row_spec(), _row_spec(), _full_spec((1, HD))],
        out_specs=[_row_spec(), _full_spec((8, HD))],
        out_shape=[
            jax.ShapeDtypeStruct((NR, HD), jnp.float32),
            jax.ShapeDtypeStruct((8, HD), jnp.float32),
        ],
    )(deg2d, acc, hp, b)


def _tc_v(t, s1):
    return pl.pallas_call(
        _tc_v_body,
        grid=(GRID,),
        in_specs=[_row_spec(), _full_spec((8, HD))],
        out_specs=_full_spec((8, HD)),
        out_shape=jax.ShapeDtypeStruct((8, HD), jnp.float32),
    )(t, s1)


def _tc_m(t, s1, s2, g, be, w):
    return pl.pallas_call(
        _tc_m_body,
        grid=(GRID,),
        in_specs=[_row_spec(), _full_spec((8, HD)), _full_spec((8, HD)),
                  _full_spec((1, HD)), _full_spec((1, HD)),
                  _full_spec((HD, HD))],
        out_specs=_row_spec(),
        out_shape=jax.ShapeDtypeStruct((NR, HD), jnp.float32),
    )(t, s1, s2, g, be, w)


def _tc_y(t, s1, s2, g, be):
    return pl.pallas_call(
        _tc_y_body,
        grid=(GRID,),
        in_specs=[_row_spec(), _full_spec((8, HD)), _full_spec((8, HD)),
                  _full_spec((1, HD)), _full_spec((1, HD))],
        out_specs=[_row_spec(), _col_spec()],
        out_shape=[
            jax.ShapeDtypeStruct((NR, HD), jnp.float32),
            jax.ShapeDtypeStruct((NR, 1), jnp.float32),
        ],
    )(t, s1, s2, g, be)


def _tc_mlp(p, w1, b1, w2, b2, w3t, b3):
    return pl.pallas_call(
        _tc_mlp_body,
        out_shape=jax.ShapeDtypeStruct((NB, 1), jnp.float32),
    )(p, w1, b1, w2, b2, w3t, b3)


def kernel(x, edge_index, batch, z_emb,
           W0, b0, g0, be0, W1, b1, g1, be1, W2, b2, g2, be2,
           mW1, mb1, mW2, mb2, mW3, mb3):
    src = edge_index[0].astype(jnp.int32)
    dst = edge_index[1].astype(jnp.int32)
    src_p = jnp.concatenate([src, jnp.zeros((EP - NE,), jnp.int32)])
    dst_p = jnp.concatenate([dst, jnp.full((EP - NE,), -1, jnp.int32)])
    batch_p = jnp.concatenate([batch.astype(jnp.int32),
                               jnp.full((NR - NN,), -1, jnp.int32)])
    x_p = jnp.concatenate([x.astype(jnp.int32),
                           jnp.zeros((NR - NN,), jnp.int32)])
    zeros1 = jnp.zeros((3200,), jnp.float32)
    zeros2 = jnp.zeros((3200, HD), jnp.float32)

    deg, counts, h0 = _sc_prep(dst_p, batch_p, x_p, z_emb, zeros1)
    deg2d = deg.reshape(NR, 1)
    dinv = _tc_dinv(deg.reshape(GRID, BR)).reshape(NR)

    ew = _sc_w(src_p, dst_p, dinv)
    hp = _tc_m0(h0, W0)
    params = ((b0, g0, be0, W1), (b1, g1, be1, W2), (b2, g2, be2, None))
    for (b, g, be, w_next) in params:
        acc = _sc_edges(src_p, dst_p, hp, ew, zeros2)
        t, s1 = _tc_t(deg2d, acc, hp, b.reshape(1, HD))
        s2 = _tc_v(t, s1)
        if w_next is not None:
            hp = _tc_m(t, s1, s2, g.reshape(1, HD), be.reshape(1, HD), w_next)
        else:
            hext, vals2d = _tc_y(t, s1, s2, g.reshape(1, HD), be.reshape(1, HD))

    cs = _tc_starts(counts)
    nodeof = _sc_rank(vals2d.reshape(NR), cs)
    nodeof_flat = nodeof.reshape(32, 16, 32).transpose(1, 0, 2).reshape(NB * 32)
    pooled = _sc_pool(hext, nodeof_flat)
    p = pooled.reshape(NB, 32, HD)[:, :KK, :].reshape(NB, KK * HD)
    o = _tc_mlp(p, mW1, mb1.reshape(1, HD), mW2, mb2.reshape(1, HD // 2),
                mW3.reshape(1, HD // 2), mb3.reshape(1, 1))
    return o[:, 0]


# speedup vs baseline: 1.0086x; 1.0086x over previous
import functools

import jax
import jax.numpy as jnp
from jax import lax
from jax.experimental import pallas as pl
from jax.experimental.pallas import tpu as pltpu
from jax.experimental.pallas import tpu_sc as plsc

NN = 100000
NE = 1600000
HD = 32
KK = 30
NB = 512
EPSV = 1e-5

BR = 4096
GRID = 25
NR = GRID * BR
EP = 1601536
CH = 128
NHALF = NN // 2
TRASH = NHALF
SENT = NN
NC, NS = 2, 16
NVBUF = 100864
NH2 = 50176

_MESH = plsc.VectorSubcoreMesh(core_axis_name="c", subcore_axis_name="s")


@functools.partial(
    pl.kernel,
    out_type=(
        jax.ShapeDtypeStruct((NR,), jnp.float32),
        jax.ShapeDtypeStruct((2, NB), jnp.float32),
        jax.ShapeDtypeStruct((NR, HD), jnp.float32),
    ),
    mesh=_MESH,
    compiler_params=pltpu.CompilerParams(use_tc_tiling_on_sc=False, needs_layout_passes=False),
    scratch_types=[
        pltpu.VMEM((CH,), jnp.int32),
        pltpu.VMEM((CH,), jnp.int32),
        pltpu.VMEM((CH,), jnp.float32),
        pltpu.VMEM((CH, HD), jnp.float32),
        pltpu.VMEM_SHARED((51200,), jnp.float32),
        pltpu.VMEM_SHARED((NB,), jnp.float32),
        pltpu.SemaphoreType.DMA,
    ],
)
def _sc_prep(dst_hbm, batch_hbm, x_hbm, zemb_hbm, zeros_hbm,
             deg_hbm, counts_hbm, h0_hbm,
             idxb, lidxb, valb, rowsb, deg_sh, cnt_sh, sem):
    cid = lax.axis_index("c")
    sid = lax.axis_index("s")
    base = cid * 51200

    pltpu.sync_copy(zeros_hbm.at[pl.ds(0, 3200)],
                    deg_sh.at[pl.ds(sid * 3200, 3200)])

    @pl.when(sid == 0)
    def _():
        pltpu.sync_copy(zeros_hbm.at[pl.ds(0, NB)], cnt_sh)

    plsc.subcore_barrier()

    ept = EP // NS
    @pl.loop(0, ept // CH)
    def _(k):
        off = sid * ept + k * CH
        pltpu.sync_copy(dst_hbm.at[pl.ds(off, CH)], idxb)
        for j in range(CH // 16):
            dd = idxb[pl.ds(j * 16, 16)]
            m = (dd >= base) & (dd < base + 51200)
            lidxb[pl.ds(j * 16, 16)] = jnp.where(m, dd - base, 0)
            valb[pl.ds(j * 16, 16)] = jnp.where(m, 1.0, 0.0)
        pltpu.sync_copy(valb, deg_sh.at[lidxb], add=True)

    @pl.loop(0, 25)
    def _(k):
        off = cid * 51200 + sid * 3200 + k * CH
        pltpu.sync_copy(batch_hbm.at[pl.ds(off, CH)], idxb)
        for j in range(CH // 16):
            bb = idxb[pl.ds(j * 16, 16)]
            m = bb >= 0
            lidxb[pl.ds(j * 16, 16)] = jnp.where(m, bb, 0)
            valb[pl.ds(j * 16, 16)] = jnp.where(m, 1.0, 0.0)
        pltpu.sync_copy(valb, cnt_sh.at[lidxb], add=True)

    wid = sid * NC + cid
    @pl.loop(0, 25)
    def _(k):
        off = wid * 3200 + k * CH
        pltpu.sync_copy(x_hbm.at[pl.ds(off, CH)], idxb)
        pltpu.async_copy(zemb_hbm.at[idxb], rowsb, sem).wait()
        pltpu.sync_copy(rowsb, h0_hbm.at[pl.ds(off, CH)])

    plsc.subcore_barrier()
    pltpu.sync_copy(deg_sh.at[pl.ds(sid * 3200, 3200)],
                    deg_hbm.at[pl.ds(base + sid * 3200, 3200)])

    @pl.when(sid == 0)
    def _():
        pltpu.sync_copy(cnt_sh, counts_hbm.at[cid])


@functools.partial(
    pl.kernel,
    out_type=jax.ShapeDtypeStruct((EP,), jnp.float32),
    mesh=_MESH,
    compiler_params=pltpu.CompilerParams(use_tc_tiling_on_sc=False, needs_layout_passes=False),
    scratch_types=[
        pltpu.VMEM((CH,), jnp.int32),
        pltpu.VMEM((CH,), jnp.int32),
        pltpu.VMEM((CH,), jnp.float32),
        pltpu.VMEM((NR,), jnp.float32),
        pltpu.SemaphoreType.DMA,
    ],
)
def _sc_w(src_hbm, dst_hbm, dinv_hbm, w_hbm, srcb, dstb, wb, dinvb, sem):
    cid = lax.axis_index("c")
    sid = lax.axis_index("s")
    wid = sid * NC + cid

    pltpu.sync_copy(dinv_hbm, dinvb)
    ept = EP // (NC * NS)
    @pl.loop(0, ept // CH)
    def _(k):
        off = wid * ept + k * CH
        cs = pltpu.async_copy(src_hbm.at[pl.ds(off, CH)], srcb, sem)
        cd = pltpu.async_copy(dst_hbm.at[pl.ds(off, CH)], dstb, sem)
        cs.wait()
        cd.wait()
        for j in range(CH // 16):
            dd = jnp.maximum(dstb[pl.ds(j * 16, 16)], 0)
            ss = srcb[pl.ds(j * 16, 16)]
            wb[pl.ds(j * 16, 16)] = (plsc.load_gather(dinvb, [ss])
                                     * plsc.load_gather(dinvb, [dd]))
        pltpu.sync_copy(wb, w_hbm.at[pl.ds(off, CH)])


NOWN = NR // 32
PCAP = 272
CHE = 512
NCH2 = EP // CHE // 2


@functools.partial(
    pl.kernel,
    out_type=jax.ShapeDtypeStruct((NR, HD), jnp.float32),
    mesh=_MESH,
    compiler_params=pltpu.CompilerParams(use_tc_tiling_on_sc=False, needs_layout_passes=False),
    scratch_types=[
        pltpu.VMEM((CHE,), jnp.int32),
        pltpu.VMEM((CHE,), jnp.int32),
        pltpu.VMEM((CHE,), jnp.float32),
        pltpu.VMEM((CHE,), jnp.int32),
        pltpu.VMEM((CHE,), jnp.int32),
        pltpu.VMEM((CHE,), jnp.float32),
        pltpu.VMEM((PCAP,), jnp.int32),
        pltpu.VMEM((PCAP,), jnp.int32),
        pltpu.VMEM((PCAP,), jnp.float32),
        pltpu.VMEM((PCAP, HD), jnp.float32),
        pltpu.VMEM((NOWN, HD), jnp.float32),
        pltpu.SemaphoreType.DMA,
        pltpu.SemaphoreType.DMA,
        pltpu.SemaphoreType.DMA,
    ],
)
def _sc_edges(src_hbm, dst_hbm, hs_hbm, w_hbm, zeros2_hbm, acc_hbm,
              srcbA, dstbA, wbA, srcbB, dstbB, wbB,
              psrc, plidx, pw, rowsb, accb, semA, semB, semG):
    cid = lax.axis_index("c")
    sid = lax.axis_index("s")
    wid = sid * NC + cid
    base = wid * NOWN

    pltpu.sync_copy(zeros2_hbm.at[pl.ds(0, NOWN)], accb)

    @pl.loop(0, PCAP // 16)
    def _(i):
        psrc[pl.ds(i * 16, 16)] = jnp.zeros((16,), jnp.int32)

    def flush(pcount):
        pltpu.async_copy(hs_hbm.at[psrc], rowsb, semG).wait()

        @pl.loop(0, pcount)
        def _(e):
            li = plidx[pl.ds(e, 16)][0]
            wf = jnp.full((16,), pw[pl.ds(e, 16)][0])
            accb[li, pl.ds(0, 16)] = (accb[li, pl.ds(0, 16)]
                                      + rowsb[e, pl.ds(0, 16)] * wf)
            accb[li, pl.ds(16, 16)] = (accb[li, pl.ds(16, 16)]
                                       + rowsb[e, pl.ds(16, 16)] * wf)

    def copy3(k, sb, db, wbuf, sem):
        off = k * CHE
        pltpu.async_copy(src_hbm.at[pl.ds(off, CHE)], sb, sem)
        pltpu.async_copy(dst_hbm.at[pl.ds(off, CHE)], db, sem)
        pltpu.async_copy(w_hbm.at[pl.ds(off, CHE)], wbuf, sem)

    def wait3(k, sb, db, wbuf, sem):
        off = k * CHE
        pltpu.make_async_copy(src_hbm.at[pl.ds(off, CHE)], sb, sem).wait()
        pltpu.make_async_copy(dst_hbm.at[pl.ds(off, CHE)], db, sem).wait()
        pltpu.make_async_copy(w_hbm.at[pl.ds(off, CHE)], wbuf, sem).wait()

    def process(sb, db, wbuf, pcount):
        for jj in range(4):
            for j in range(8):
                q16 = (jj * 8 + j) * 16
                dd = db[pl.ds(q16, 16)]
                m = (dd >= base) & (dd < base + NOWN)
                plsc.store_compressed(psrc.at[pl.ds(pcount, 16)],
                                      sb[pl.ds(q16, 16)], mask=m)
                plsc.store_compressed(plidx.at[pl.ds(pcount, 16)], dd - base,
                                      mask=m)
                plsc.store_compressed(pw.at[pl.ds(pcount, 16)],
                                      wbuf[pl.ds(q16, 16)], mask=m)
                pcount = pcount + plsc.all_reduce_population_count(m)[0]

            @pl.when(pcount >= CH)
            def _():
                flush(pcount)

            pcount = jnp.where(pcount >= CH, 0, pcount)
        return pcount

    copy3(0, srcbA, dstbA, wbA, semA)

    @pl.loop(0, NCH2, init_carry=jnp.int32(0))
    def _(q, pcount):
        kA = 2 * q
        copy3(kA + 1, srcbB, dstbB, wbB, semB)
        wait3(kA, srcbA, dstbA, wbA, semA)
        pcount = process(srcbA, dstbA, wbA, pcount)

        @pl.when(q + 1 < NCH2)
        def _():
            copy3(kA + 2, srcbA, dstbA, wbA, semA)

        wait3(kA + 1, srcbB, dstbB, wbB, semB)
        pcount = process(srcbB, dstbB, wbB, pcount)
        return pcount

    flush(_)

    pltpu.sync_copy(accb, acc_hbm.at[pl.ds(base, NOWN)])


@functools.partial(
    pl.kernel,
    out_type=jax.ShapeDtypeStruct((32, 512), jnp.int32),
    mesh=_MESH,
    compiler_params=pltpu.CompilerParams(use_tc_tiling_on_sc=False, needs_layout_passes=False),
    scratch_types=[
        pltpu.VMEM((NVBUF,), jnp.float32),
        pltpu.VMEM((NB,), jnp.float32),
        pltpu.VMEM((NB,), jnp.float32),
        pltpu.VMEM((NB + 32,), jnp.int32),
        pltpu.VMEM((NB + 32,), jnp.int32),
        pltpu.VMEM((NB,), jnp.int32),
        pltpu.SemaphoreType.DMA,
    ],
)
def _sc_rank(vals_hbm, cs_hbm, nodeof_hbm,
             vbuf, cp0, cp1, cntb, startsb, slotb, sem):
    cid = lax.axis_index("c")
    sid = lax.axis_index("s")
    wid = sid * NC + cid

    pltpu.sync_copy(cs_hbm.at[0], cp0)
    pltpu.sync_copy(cs_hbm.at[1], cp1)

    @pl.loop(0, NB // 16)
    def _(i):
        cntb[pl.ds(i * 16, 16)] = cp0[pl.ds(i * 16, 16)].astype(jnp.int32)
        startsb[pl.ds(i * 16, 16)] = cp1[pl.ds(i * 16, 16)].astype(jnp.int32)

    lanes = lax.iota(jnp.int32, 16)

    @pl.loop(0, NB // 32)
    def _(t):
        g = t * 32 + wid
        st = startsb[pl.ds(g, 16)][0]
        c = cntb[pl.ds(g, 16)][0]
        s8 = (st // 8) * 8
        off = st - s8
        nch = (off + c + 511) // 512

        @pl.loop(0, nch)
        def _(k):
            pltpu.sync_copy(vals_hbm.at[pl.ds(s8 + k * 512, 512)],
                            vbuf.at[pl.ds(k * 512, 512)])

        slotb[pl.ds(t * 32, 16)] = jnp.full((16,), SENT, jnp.int32)
        slotb[pl.ds(t * 32 + 16, 16)] = jnp.full((16,), SENT, jnp.int32)

        @pl.loop(0, (c + 15) // 16)
        def _(ic):
            ivec = lanes + ic * 16
            vi = vbuf[pl.ds(off + ic * 16, 16)]

            def body_jc(jc, rank):
                vj16 = vbuf[pl.ds(off + jc * 16, 16)]
                for jj in range(16):
                    vjs = jnp.full((16,), vj16[jj])
                    jidx = jc * 16 + jj
                    valid = jidx < c
                    gt = (vjs > vi) & valid
                    eq = (vjs == vi) & (jidx < ivec) & valid
                    rank = rank + jnp.where(gt, 1, 0) + jnp.where(eq, 1, 0)
                return rank

            rank = lax.fori_loop(0, (c + 15) // 16, body_jc,
                                 jnp.zeros((16,), jnp.int32))
            m = (ivec < c) & (rank < KK)
            plsc.store_scatter(slotb, [t * 32 + jnp.minimum(rank, 31)],
                               st + ivec, mask=m)

    pltpu.sync_copy(slotb, nodeof_hbm.at[wid])


@functools.partial(
    pl.kernel,
    out_type=jax.ShapeDtypeStruct((NB * 32, HD), jnp.float32),
    mesh=_MESH,
    compiler_params=pltpu.CompilerParams(use_tc_tiling_on_sc=False, needs_layout_passes=False),
    scratch_types=[
        pltpu.VMEM((CH,), jnp.int32),
        pltpu.VMEM((CH, HD), jnp.float32),
        pltpu.SemaphoreType.DMA,
    ],
)
def _sc_pool(hext_hbm, nodeof_hbm, pooled_hbm, idxb, rowsb, sem):
    cid = lax.axis_index("c")
    sid = lax.axis_index("s")
    wid = sid * NC + cid

    @pl.loop(0, (NB * 32) // (32 * CH))
    def _(k):
        off = wid * 512 + k * CH
        pltpu.sync_copy(nodeof_hbm.at[pl.ds(off, CH)], idxb)
        pltpu.async_copy(hext_hbm.at[idxb], rowsb, sem).wait()
        pltpu.sync_copy(rowsb, pooled_hbm.at[pl.ds(off, CH)])


def _row_spec():
    return pl.BlockSpec((BR, HD), lambda i: (i, 0))


def _col_spec():
    return pl.BlockSpec((BR, 1), lambda i: (i, 0))


def _full_spec(shape):
    return pl.BlockSpec(shape, lambda i: tuple(0 for _ in shape))


def _tc_dinv_body(deg_ref, dinv_ref):
    dinv_ref[...] = lax.rsqrt(deg_ref[...] + 1.0)


def _tc_m0_body(h0_ref, w_ref, hs_ref):
    hs_ref[...] = jnp.dot(h0_ref[...], w_ref[...],
                          preferred_element_type=jnp.float32)


def _tc_t_body(deg_ref, acc_ref, hp_ref, b_ref, t_ref, s1_ref):
    i = pl.program_id(0)
    dinv = lax.rsqrt(deg_ref[...] + 1.0)
    t = acc_ref[...] + hp_ref[...] * (dinv * dinv) + b_ref[...]
    t_ref[...] = t
    rows = lax.broadcasted_iota(jnp.int32, (BR, 1), 0) + i * BR
    tm = jnp.where(rows < NN, t, 0.0)

    @pl.when(i == 0)
    def _():
        s1_ref[...] = jnp.zeros((8, HD), jnp.float32)

    s1_ref[0:1, :] = s1_ref[0:1, :] + jnp.sum(tm, axis=0, keepdims=True)


def _tc_v_body(t_ref, s1_ref, s2_ref):
    i = pl.program_id(0)
    mean = s1_ref[0:1, :] / NN
    dev = t_ref[...] - mean
    rows = lax.broadcasted_iota(jnp.int32, (BR, 1), 0) + i * BR
    dev = jnp.where(rows < NN, dev, 0.0)

    @pl.when(i == 0)
    def _():
        s2_ref[...] = jnp.zeros((8, HD), jnp.float32)

    s2_ref[0:1, :] = s2_ref[0:1, :] + jnp.sum(dev * dev, axis=0, keepdims=True)


def _tc_m_body(t_ref, s1_ref, s2_ref, g_ref, be_ref, w_ref, hs_ref):
    mean = s1_ref[0:1, :] / NN
    var = s2_ref[0:1, :] / NN
    y = (t_ref[...] - mean) * lax.rsqrt(var + EPSV) * g_ref[...] + be_ref[...]
    y = jnp.maximum(y, 0.0)
    hs_ref[...] = jnp.dot(y, w_ref[...], preferred_element_type=jnp.float32)


def _tc_y_body(t_ref, s1_ref, s2_ref, g_ref, be_ref, hext_ref, vals_ref):
    i = pl.program_id(0)
    mean = s1_ref[0:1, :] / NN
    var = s2_ref[0:1, :] / NN
    y = (t_ref[...] - mean) * lax.rsqrt(var + EPSV) * g_ref[...] + be_ref[...]
    y = jnp.maximum(y, 0.0)
    rows = lax.broadcasted_iota(jnp.int32, (BR, 1), 0) + i * BR
    y = jnp.where(rows < NN, y, 0.0)
    hext_ref[...] = y
    vals_ref[...] = y[:, HD - 1:HD]


def _tc_starts_body(c_ref, cs_ref):
    cnt = c_ref[0:1, :] + c_ref[1:2, :]
    ii = lax.broadcasted_iota(jnp.int32, (NB, NB), 0)
    jj = lax.broadcasted_iota(jnp.int32, (NB, NB), 1)
    lt = jnp.where(ii < jj, 1.0, 0.0)
    starts = jnp.dot(cnt, lt, preferred_element_type=jnp.float32)
    cs_ref[0:1, :] = cnt
    cs_ref[1:2, :] = starts


def _tc_starts(counts):
    return pl.pallas_call(
        _tc_starts_body,
        out_shape=jax.ShapeDtypeStruct((2, NB), jnp.float32),
    )(counts)


def _tc_mlp_body(p_ref, w1_ref, b1_ref, w2_ref, b2_ref, w3t_ref, b3_ref, o_ref):
    h2 = jnp.dot(p_ref[...], w1_ref[...], preferred_element_type=jnp.float32)
    h2 = jnp.maximum(h2 + b1_ref[...], 0.0)
    h3 = jnp.dot(h2, w2_ref[...], preferred_element_type=jnp.float32)
    h3 = jnp.maximum(h3 + b2_ref[...], 0.0)
    o = jnp.sum(h3 * w3t_ref[...], axis=1, keepdims=True) + b3_ref[...]
    o_ref[...] = o


def _tc_dinv(deg25):
    return pl.pallas_call(
        _tc_dinv_body,
        out_shape=jax.ShapeDtypeStruct((GRID, BR), jnp.float32),
    )(deg25)


def _tc_m0(h0, w):
    return pl.pallas_call(
        _tc_m0_body,
        grid=(GRID,),
        in_specs=[_row_spec(), _full_spec((HD, HD))],
        out_specs=_row_spec(),
        out_shape=jax.ShapeDtypeStruct((NR, HD), jnp.float32),
    )(h0, w)


def _tc_t(deg2d, acc, hp, b):
    return pl.pallas_call(
        _tc_t_body,
        grid=(GRID,),
        in_specs=[_col_spec(), _row_spec(), _row_spec(), _full_spec((1, HD))],
        out_specs=[_row_spec(), _full_spec((8, HD))],
        out_shape=[
            jax.ShapeDtypeStruct((NR, HD), jnp.float32),
            jax.ShapeDtypeStruct((8, HD), jnp.float32),
        ],
    )(deg2d, acc, hp, b)


def _tc_v(t, s1):
    return pl.pallas_call(
        _tc_v_body,
        grid=(GRID,),
        in_specs=[_row_spec(), _full_spec((8, HD))],
        out_specs=_full_spec((8, HD)),
        out_shape=jax.ShapeDtypeStruct((8, HD), jnp.float32),
    )(t, s1)


def _tc_m(t, s1, s2, g, be, w):
    return pl.pallas_call(
        _tc_m_body,
        grid=(GRID,),
        in_specs=[_row_spec(), _full_spec((8, HD)), _full_spec((8, HD)),
                  _full_spec((1, HD)), _full_spec((1, HD)),
                  _full_spec((HD, HD))],
        out_specs=_row_spec(),
        out_shape=jax.ShapeDtypeStruct((NR, HD), jnp.float32),
    )(t, s1, s2, g, be, w)


def _tc_y(t, s1, s2, g, be):
    return pl.pallas_call(
        _tc_y_body,
        grid=(GRID,),
        in_specs=[_row_spec(), _full_spec((8, HD)), _full_spec((8, HD)),
                  _full_spec((1, HD)), _full_spec((1, HD))],
        out_specs=[_row_spec(), _col_spec()],
        out_shape=[
            jax.ShapeDtypeStruct((NR, HD), jnp.float32),
            jax.ShapeDtypeStruct((NR, 1), jnp.float32),
        ],
    )(t, s1, s2, g, be)


def _tc_mlp(p, w1, b1, w2, b2, w3t, b3):
    return pl.pallas_call(
        _tc_mlp_body,
        out_shape=jax.ShapeDtypeStruct((NB, 1), jnp.float32),
    )(p, w1, b1, w2, b2, w3t, b3)


def kernel(x, edge_index, batch, z_emb,
           W0, b0, g0, be0, W1, b1, g1, be1, W2, b2, g2, be2,
           mW1, mb1, mW2, mb2, mW3, mb3):
    src = edge_index[0].astype(jnp.int32)
    dst = edge_index[1].astype(jnp.int32)
    src_p = jnp.concatenate([src, jnp.zeros((EP - NE,), jnp.int32)])
    dst_p = jnp.concatenate([dst, jnp.full((EP - NE,), -1, jnp.int32)])
    batch_p = jnp.concatenate([batch.astype(jnp.int32),
                               jnp.full((NR - NN,), -1, jnp.int32)])
    x_p = jnp.concatenate([x.astype(jnp.int32),
                           jnp.zeros((NR - NN,), jnp.int32)])
    zeros1 = jnp.zeros((3200,), jnp.float32)
    zeros2 = jnp.zeros((3200, HD), jnp.float32)

    deg, counts, h0 = _sc_prep(dst_p, batch_p, x_p, z_emb, zeros1)
    deg2d = deg.reshape(NR, 1)
    dinv = _tc_dinv(deg.reshape(GRID, BR)).reshape(NR)

    ew = _sc_w(src_p, dst_p, dinv)
    hp = _tc_m0(h0, W0)
    params = ((b0, g0, be0, W1), (b1, g1, be1, W2), (b2, g2, be2, None))
    for (b, g, be, w_next) in params:
        acc = _sc_edges(src_p, dst_p, hp, ew, zeros2)
        t, s1 = _tc_t(deg2d, acc, hp, b.reshape(1, HD))
        s2 = _tc_v(t, s1)
        if w_next is not None:
            hp = _tc_m(t, s1, s2, g.reshape(1, HD), be.reshape(1, HD), w_next)
        else:
            hext, vals2d = _tc_y(t, s1, s2, g.reshape(1, HD), be.reshape(1, HD))

    cs = _tc_starts(counts)
    nodeof = _sc_rank(vals2d.reshape(NR), cs)
    nodeof_flat = nodeof.reshape(32, 16, 32).transpose(1, 0, 2).reshape(NB * 32)
    pooled = _sc_pool(hext, nodeof_flat)
    p = pooled.reshape(NB, 32, HD)[:, :KK, :].reshape(NB, KK * HD)
    o = _tc_mlp(p, mW1, mb1.reshape(1, HD), mW2, mb2.reshape(1, HD // 2),
                mW3.reshape(1, HD // 2), mb3.reshape(1, 1))
    return o[:, 0]
